# Initial kernel scaffold; baseline (speedup 1.0000x reference)
import functools

import jax
import jax.numpy as jnp
from jax import lax
from jax.experimental import pallas as pl
from jax.experimental.pallas import tpu as pltpu
from jax.experimental.pallas import tpu_sc as plsc

B = 16384
L = 200
V = 1000000
D = 32
H = 64
NLAB = 10

NW = 32
SPW = B // NW
G = 32
NCH = SPW // G
VL = 16


def _sc_pool(shp_flat, col_flat, shape_table, color_table):
    mesh = plsc.VectorSubcoreMesh(core_axis_name="c", subcore_axis_name="s")

    @functools.partial(
        pl.kernel,
        out_type=jax.ShapeDtypeStruct((B, 2 * D), jnp.float32),
        mesh=mesh,
        scratch_types=[
            pltpu.VMEM((G * L,), jnp.int32),
            pltpu.VMEM((G * L,), jnp.int32),
            pltpu.VMEM((L, D), jnp.float32),
            pltpu.VMEM((L, D), jnp.float32),
            pltpu.VMEM((G, 2 * D), jnp.float32),
            pltpu.SemaphoreType.DMA,
            pltpu.SemaphoreType.DMA,
        ],
    )
    def pool(shp_hbm, col_hbm, stab_hbm, ctab_hbm, out_hbm,
             idx_s, idx_c, rows_s, rows_c, feat_v, sem_s, sem_c):
        wid = lax.axis_index("s") * 2 + lax.axis_index("c")
        wbase = wid * SPW

        def chunk_body(ch, _):
            ibase = pl.multiple_of((wbase + ch * G) * L, 8)
            pltpu.sync_copy(shp_hbm.at[pl.ds(ibase, G * L)], idx_s)
            pltpu.sync_copy(col_hbm.at[pl.ds(ibase, G * L)], idx_c)

            def mask_body(i, _):
                sl = pl.ds(pl.multiple_of(i * VL, 8), VL)
                s = idx_s[sl]
                c = idx_c[sl]
                idx_c[sl] = jnp.where(s == 0, 0, c)
                return 0

            lax.fori_loop(0, G * L // VL, mask_body, 0, unroll=4)

            def sample_body(g, _):
                off = pl.multiple_of(g * L, 8)
                off2 = pl.multiple_of(g * L + 128, 8)
                cp1 = pltpu.async_copy(
                    stab_hbm.at[idx_s.at[pl.ds(off, 128)]],
                    rows_s.at[pl.ds(0, 128)], sem_s)
                cp2 = pltpu.async_copy(
                    stab_hbm.at[idx_s.at[pl.ds(off2, L - 128)]],
                    rows_s.at[pl.ds(128, L - 128)], sem_s)
                cp3 = pltpu.async_copy(
                    ctab_hbm.at[idx_c.at[pl.ds(off, 128)]],
                    rows_c.at[pl.ds(0, 128)], sem_c)
                cp4 = pltpu.async_copy(
                    ctab_hbm.at[idx_c.at[pl.ds(off2, L - 128)]],
                    rows_c.at[pl.ds(128, L - 128)], sem_c)
                cp1.wait()
                cp2.wait()
                cp3.wait()
                cp4.wait()

                def acc_body(l, carry):
                    a0, a1, c0, c1 = carry
                    a0 = a0 + rows_s[l, pl.ds(0, VL)]
                    a1 = a1 + rows_s[l, pl.ds(VL, VL)]
                    c0 = c0 + rows_c[l, pl.ds(0, VL)]
                    c1 = c1 + rows_c[l, pl.ds(VL, VL)]
                    return (a0, a1, c0, c1)

                z = jnp.zeros((VL,), jnp.float32)
                a0, a1, c0, c1 = lax.fori_loop(0, L, acc_body, (z, z, z, z),
                                               unroll=4)
                feat_v[g, pl.ds(0, VL)] = a0
                feat_v[g, pl.ds(VL, VL)] = a1
                feat_v[g, pl.ds(2 * VL, VL)] = c0
                feat_v[g, pl.ds(3 * VL, VL)] = c1
                return 0

            lax.fori_loop(0, G, sample_body, 0)
            pltpu.sync_copy(feat_v, out_hbm.at[pl.ds(wbase + ch * G, G)])
            return 0

        lax.fori_loop(0, NCH, chunk_body, 0)

    return pool(shp_flat, col_flat, shape_table, color_table)


def _mlp(feat_raw, lens_col, W1, b1, W2, b2):
    Bb = 2048

    def body(feat_ref, lens_ref, w1_ref, b1_ref, w2_ref, b2_ref, out_ref):
        f = feat_ref[...]
        inv = 1.0 / lens_ref[...]
        h = jnp.dot(f, w1_ref[...], preferred_element_type=jnp.float32)
        h = jnp.maximum(h * inv + b1_ref[...], 0.0)
        out_ref[...] = (
            jnp.dot(h, w2_ref[...], preferred_element_type=jnp.float32)
            + b2_ref[...]
        )

    return pl.pallas_call(
        body,
        grid=(B // Bb,),
        in_specs=[
            pl.BlockSpec((Bb, 2 * D), lambda i: (i, 0)),
            pl.BlockSpec((Bb, 1), lambda i: (i, 0)),
            pl.BlockSpec((2 * D, H), lambda i: (0, 0)),
            pl.BlockSpec((1, H), lambda i: (0, 0)),
            pl.BlockSpec((H, NLAB), lambda i: (0, 0)),
            pl.BlockSpec((1, NLAB), lambda i: (0, 0)),
        ],
        out_specs=pl.BlockSpec((Bb, NLAB), lambda i: (i, 0)),
        out_shape=jax.ShapeDtypeStruct((B, NLAB), jnp.float32),
    )(feat_raw, lens_col, W1, b1.reshape(1, H), W2, b2.reshape(1, NLAB))


def kernel(shp_ids, col_ids, lens, shape_table, color_table, W1, b1, W2, b2):
    feat = _sc_pool(shp_ids.reshape(-1), col_ids.reshape(-1),
                    shape_table, color_table)
    lens_col = lens.astype(jnp.float32).reshape(B, 1)
    return _mlp(feat, lens_col, W1, b1, W2, b2)


# speedup vs baseline: 11.5897x; 11.5897x over previous
import functools

import jax
import jax.numpy as jnp
from jax import lax
from jax.experimental import pallas as pl
from jax.experimental.pallas import tpu as pltpu
from jax.experimental.pallas import tpu_sc as plsc

B = 16384
L = 200
V = 1000000
D = 32
H = 64
NLAB = 10

NW = 32
SPW = B // NW
G = 32
NCH = SPW // G
VL = 16


def _sc_pool(shp_flat, col_flat, shape_table, color_table):
    mesh = plsc.VectorSubcoreMesh(core_axis_name="c", subcore_axis_name="s")

    @functools.partial(
        pl.kernel,
        out_type=jax.ShapeDtypeStruct((B, 2 * D), jnp.float32),
        mesh=mesh,
        compiler_params=pltpu.CompilerParams(use_tc_tiling_on_sc=False),
        scratch_types=[
            pltpu.VMEM((G * L,), jnp.int32),
            pltpu.VMEM((G * L,), jnp.int32),
            pltpu.VMEM((L, D), jnp.float32),
            pltpu.VMEM((L, D), jnp.float32),
            pltpu.VMEM((G, 2 * D), jnp.float32),
            pltpu.SemaphoreType.DMA,
            pltpu.SemaphoreType.DMA,
        ],
    )
    def pool(shp_hbm, col_hbm, stab_hbm, ctab_hbm, out_hbm,
             idx_s, idx_c, rows_s, rows_c, feat_v, sem_s, sem_c):
        wid = lax.axis_index("s") * 2 + lax.axis_index("c")
        wbase = wid * SPW

        def chunk_body(ch, _):
            ibase = pl.multiple_of((wbase + ch * G) * L, 8)
            pltpu.sync_copy(shp_hbm.at[pl.ds(ibase, G * L)], idx_s)
            pltpu.sync_copy(col_hbm.at[pl.ds(ibase, G * L)], idx_c)

            def mask_body(i, _):
                sl = pl.ds(pl.multiple_of(i * VL, 8), VL)
                s = idx_s[sl]
                c = idx_c[sl]
                idx_c[sl] = jnp.where(s == 0, 0, c)
                return 0

            lax.fori_loop(0, G * L // VL, mask_body, 0, unroll=4)

            def sample_body(g, _):
                off = pl.multiple_of(g * L, 8)
                off2 = pl.multiple_of(g * L + 128, 8)
                cp1 = pltpu.async_copy(
                    stab_hbm.at[idx_s.at[pl.ds(off, 128)]],
                    rows_s.at[pl.ds(0, 128)], sem_s)
                cp2 = pltpu.async_copy(
                    stab_hbm.at[idx_s.at[pl.ds(off2, L - 128)]],
                    rows_s.at[pl.ds(128, L - 128)], sem_s)
                cp3 = pltpu.async_copy(
                    ctab_hbm.at[idx_c.at[pl.ds(off, 128)]],
                    rows_c.at[pl.ds(0, 128)], sem_c)
                cp4 = pltpu.async_copy(
                    ctab_hbm.at[idx_c.at[pl.ds(off2, L - 128)]],
                    rows_c.at[pl.ds(128, L - 128)], sem_c)
                cp1.wait()
                cp2.wait()
                cp3.wait()
                cp4.wait()

                def acc_body(l, carry):
                    a0, a1, c0, c1 = carry
                    a0 = a0 + rows_s[l, pl.ds(0, VL)]
                    a1 = a1 + rows_s[l, pl.ds(VL, VL)]
                    c0 = c0 + rows_c[l, pl.ds(0, VL)]
                    c1 = c1 + rows_c[l, pl.ds(VL, VL)]
                    return (a0, a1, c0, c1)

                z = jnp.zeros((VL,), jnp.float32)
                a0, a1, c0, c1 = lax.fori_loop(0, L, acc_body, (z, z, z, z),
                                               unroll=4)
                feat_v[g, pl.ds(0, VL)] = a0
                feat_v[g, pl.ds(VL, VL)] = a1
                feat_v[g, pl.ds(2 * VL, VL)] = c0
                feat_v[g, pl.ds(3 * VL, VL)] = c1
                return 0

            lax.fori_loop(0, G, sample_body, 0)
            pltpu.sync_copy(feat_v, out_hbm.at[pl.ds(wbase + ch * G, G)])
            return 0

        lax.fori_loop(0, NCH, chunk_body, 0)

    return pool(shp_flat, col_flat, shape_table, color_table)


def _mlp(feat_raw, lens_col, W1, b1, W2, b2):
    Bb = 2048

    def body(feat_ref, lens_ref, w1_ref, b1_ref, w2_ref, b2_ref, out_ref):
        f = feat_ref[...]
        inv = 1.0 / lens_ref[...]
        h = jnp.dot(f, w1_ref[...], preferred_element_type=jnp.float32)
        h = jnp.maximum(h * inv + b1_ref[...], 0.0)
        out_ref[...] = (
            jnp.dot(h, w2_ref[...], preferred_element_type=jnp.float32)
            + b2_ref[...]
        )

    return pl.pallas_call(
        body,
        grid=(B // Bb,),
        in_specs=[
            pl.BlockSpec((Bb, 2 * D), lambda i: (i, 0)),
            pl.BlockSpec((Bb, 1), lambda i: (i, 0)),
            pl.BlockSpec((2 * D, H), lambda i: (0, 0)),
            pl.BlockSpec((1, H), lambda i: (0, 0)),
            pl.BlockSpec((H, NLAB), lambda i: (0, 0)),
            pl.BlockSpec((1, NLAB), lambda i: (0, 0)),
        ],
        out_specs=pl.BlockSpec((Bb, NLAB), lambda i: (i, 0)),
        out_shape=jax.ShapeDtypeStruct((B, NLAB), jnp.float32),
    )(feat_raw, lens_col, W1, b1.reshape(1, H), W2, b2.reshape(1, NLAB))


def kernel(shp_ids, col_ids, lens, shape_table, color_table, W1, b1, W2, b2):
    feat = _sc_pool(shp_ids.reshape(-1), col_ids.reshape(-1),
                    shape_table, color_table)
    lens_col = lens.astype(jnp.float32).reshape(B, 1)
    return _mlp(feat, lens_col, W1, b1, W2, b2)


# speedup vs baseline: 14.4030x; 1.2427x over previous
import functools

import jax
import jax.numpy as jnp
from jax import lax
from jax.experimental import pallas as pl
from jax.experimental.pallas import tpu as pltpu
from jax.experimental.pallas import tpu_sc as plsc

B = 16384
L = 200
V = 1000000
D = 32
H = 64
NLAB = 10

NW = 32
SPW = B // NW
G = 32
NCH = SPW // G
VL = 16


def _sc_pool(shp_flat, col_flat, shape_table, color_table):
    mesh = plsc.VectorSubcoreMesh(core_axis_name="c", subcore_axis_name="s")

    @functools.partial(
        pl.kernel,
        out_type=jax.ShapeDtypeStruct((B, 2 * D), jnp.float32),
        mesh=mesh,
        compiler_params=pltpu.CompilerParams(use_tc_tiling_on_sc=False),
        scratch_types=[
            pltpu.VMEM((G * L,), jnp.int32),
            pltpu.VMEM((G * L,), jnp.int32),
            pltpu.VMEM((L, D), jnp.float32),
            pltpu.VMEM((L, D), jnp.float32),
            pltpu.VMEM((L, D), jnp.float32),
            pltpu.VMEM((L, D), jnp.float32),
            pltpu.VMEM((G, 2 * D), jnp.float32),
            pltpu.SemaphoreType.DMA,
            pltpu.SemaphoreType.DMA,
            pltpu.SemaphoreType.DMA,
            pltpu.SemaphoreType.DMA,
        ],
    )
    def pool(shp_hbm, col_hbm, stab_hbm, ctab_hbm, out_hbm,
             idx_s, idx_c, rows_s0, rows_s1, rows_c0, rows_c1, feat_v,
             sem_s0, sem_s1, sem_c0, sem_c1):
        wid = lax.axis_index("s") * 2 + lax.axis_index("c")
        wbase = wid * SPW
        bufs = ((rows_s0, rows_c0, sem_s0, sem_c0),
                (rows_s1, rows_c1, sem_s1, sem_c1))

        def descs(g, b):
            rs, rc, ss, sc = bufs[b]
            off = pl.multiple_of(g * L, 8)
            off2 = pl.multiple_of(g * L + 128, 8)
            return (
                pltpu.make_async_copy(
                    stab_hbm.at[idx_s.at[pl.ds(off, 128)]],
                    rs.at[pl.ds(0, 128)], ss),
                pltpu.make_async_copy(
                    stab_hbm.at[idx_s.at[pl.ds(off2, L - 128)]],
                    rs.at[pl.ds(128, L - 128)], ss),
                pltpu.make_async_copy(
                    ctab_hbm.at[idx_c.at[pl.ds(off, 128)]],
                    rc.at[pl.ds(0, 128)], sc),
                pltpu.make_async_copy(
                    ctab_hbm.at[idx_c.at[pl.ds(off2, L - 128)]],
                    rc.at[pl.ds(128, L - 128)], sc),
            )

        def issue(g, b):
            for d in descs(g, b):
                d.start()

        def wait(g, b):
            for d in descs(g, b):
                d.wait()

        def accumulate(g, b):
            rs, rc, _, _ = bufs[b]

            def acc_body(l, carry):
                a0, a1, c0, c1 = carry
                a0 = a0 + rs[l, pl.ds(0, VL)]
                a1 = a1 + rs[l, pl.ds(VL, VL)]
                c0 = c0 + rc[l, pl.ds(0, VL)]
                c1 = c1 + rc[l, pl.ds(VL, VL)]
                return (a0, a1, c0, c1)

            z = jnp.zeros((VL,), jnp.float32)
            a0, a1, c0, c1 = lax.fori_loop(0, L, acc_body, (z, z, z, z),
                                           unroll=4)
            feat_v[g, pl.ds(0, VL)] = a0
            feat_v[g, pl.ds(VL, VL)] = a1
            feat_v[g, pl.ds(2 * VL, VL)] = c0
            feat_v[g, pl.ds(3 * VL, VL)] = c1

        def chunk_body(ch, _):
            ibase = pl.multiple_of((wbase + ch * G) * L, 8)
            pltpu.sync_copy(shp_hbm.at[pl.ds(ibase, G * L)], idx_s)
            pltpu.sync_copy(col_hbm.at[pl.ds(ibase, G * L)], idx_c)

            def mask_body(i, _):
                sl = pl.ds(pl.multiple_of(i * VL, 8), VL)
                s = idx_s[sl]
                c = idx_c[sl]
                idx_c[sl] = jnp.where(s == 0, 0, c)
                return 0

            lax.fori_loop(0, G * L // VL, mask_body, 0, unroll=4)

            issue(0, 0)
            issue(1, 1)

            def pair_body(pp, _):
                for b in range(2):
                    g = pp * 2 + b
                    wait(g, b)
                    accumulate(g, b)
                    issue(g + 2, b)
                return 0

            lax.fori_loop(0, G // 2 - 1, pair_body, 0)
            for b in range(2):
                g = G - 2 + b
                wait(g, b)
                accumulate(g, b)

            pltpu.sync_copy(feat_v, out_hbm.at[pl.ds(wbase + ch * G, G)])
            return 0

        lax.fori_loop(0, NCH, chunk_body, 0)

    return pool(shp_flat, col_flat, shape_table, color_table)


def _mlp(feat_raw, lens_col, W1, b1, W2, b2):
    Bb = 2048

    def body(feat_ref, lens_ref, w1_ref, b1_ref, w2_ref, b2_ref, out_ref):
        f = feat_ref[...]
        inv = 1.0 / lens_ref[...]
        h = jnp.dot(f, w1_ref[...], preferred_element_type=jnp.float32)
        h = jnp.maximum(h * inv + b1_ref[...], 0.0)
        out_ref[...] = (
            jnp.dot(h, w2_ref[...], preferred_element_type=jnp.float32)
            + b2_ref[...]
        )

    return pl.pallas_call(
        body,
        grid=(B // Bb,),
        in_specs=[
            pl.BlockSpec((Bb, 2 * D), lambda i: (i, 0)),
            pl.BlockSpec((Bb, 1), lambda i: (i, 0)),
            pl.BlockSpec((2 * D, H), lambda i: (0, 0)),
            pl.BlockSpec((1, H), lambda i: (0, 0)),
            pl.BlockSpec((H, NLAB), lambda i: (0, 0)),
            pl.BlockSpec((1, NLAB), lambda i: (0, 0)),
        ],
        out_specs=pl.BlockSpec((Bb, NLAB), lambda i: (i, 0)),
        out_shape=jax.ShapeDtypeStruct((B, NLAB), jnp.float32),
    )(feat_raw, lens_col, W1, b1.reshape(1, H), W2, b2.reshape(1, NLAB))


def kernel(shp_ids, col_ids, lens, shape_table, color_table, W1, b1, W2, b2):
    feat = _sc_pool(shp_ids.reshape(-1), col_ids.reshape(-1),
                    shape_table, color_table)
    lens_col = lens.astype(jnp.float32).reshape(B, 1)
    return _mlp(feat, lens_col, W1, b1, W2, b2)


# speedup vs baseline: 14.6155x; 1.0148x over previous
import functools

import jax
import jax.numpy as jnp
from jax import lax
from jax.experimental import pallas as pl
from jax.experimental.pallas import tpu as pltpu
from jax.experimental.pallas import tpu_sc as plsc

B = 16384
L = 200
V = 1000000
D = 32
H = 64
NLAB = 10

NW = 32
SPW = B // NW
G = 32
NCH = SPW // G
VL = 16


def _sc_pool(shp_ids, col_ids, shape_table, color_table):
    mesh = plsc.VectorSubcoreMesh(core_axis_name="c", subcore_axis_name="s")

    @functools.partial(
        pl.kernel,
        out_type=jax.ShapeDtypeStruct((B, 2 * D), jnp.float32),
        mesh=mesh,
        compiler_params=pltpu.CompilerParams(use_tc_tiling_on_sc=False),
        scratch_types=[
            pltpu.VMEM((G, L), jnp.int32),
            pltpu.VMEM((G, L), jnp.int32),
            pltpu.VMEM((L, D), jnp.float32),
            pltpu.VMEM((L, D), jnp.float32),
            pltpu.VMEM((L, D), jnp.float32),
            pltpu.VMEM((L, D), jnp.float32),
            pltpu.VMEM((G, 2 * D), jnp.float32),
            pltpu.SemaphoreType.DMA,
            pltpu.SemaphoreType.DMA,
            pltpu.SemaphoreType.DMA,
            pltpu.SemaphoreType.DMA,
        ],
    )
    def pool(shp_hbm, col_hbm, stab_hbm, ctab_hbm, out_hbm,
             idx_s, idx_c, rows_s0, rows_s1, rows_c0, rows_c1, feat_v,
             sem_s0, sem_s1, sem_c0, sem_c1):
        wid = lax.axis_index("s") * 2 + lax.axis_index("c")
        wbase = wid * SPW
        bufs = ((rows_s0, rows_c0, sem_s0, sem_c0),
                (rows_s1, rows_c1, sem_s1, sem_c1))

        def descs(g, b):
            rs, rc, ss, sc = bufs[b]
            return (
                pltpu.make_async_copy(
                    stab_hbm.at[idx_s.at[g, pl.ds(0, 128)]],
                    rs.at[pl.ds(0, 128)], ss),
                pltpu.make_async_copy(
                    stab_hbm.at[idx_s.at[g, pl.ds(128, L - 128)]],
                    rs.at[pl.ds(128, L - 128)], ss),
                pltpu.make_async_copy(
                    ctab_hbm.at[idx_c.at[g, pl.ds(0, 128)]],
                    rc.at[pl.ds(0, 128)], sc),
                pltpu.make_async_copy(
                    ctab_hbm.at[idx_c.at[g, pl.ds(128, L - 128)]],
                    rc.at[pl.ds(128, L - 128)], sc),
            )

        def issue(g, b):
            for d in descs(g, b):
                d.start()

        def wait(g, b):
            for d in descs(g, b):
                d.wait()

        def accumulate(g, b):
            rs, rc, _, _ = bufs[b]

            def acc_body(l, carry):
                a0, a1, c0, c1 = carry
                a0 = a0 + rs[l, pl.ds(0, VL)]
                a1 = a1 + rs[l, pl.ds(VL, VL)]
                c0 = c0 + rc[l, pl.ds(0, VL)]
                c1 = c1 + rc[l, pl.ds(VL, VL)]
                return (a0, a1, c0, c1)

            z = jnp.zeros((VL,), jnp.float32)
            a0, a1, c0, c1 = lax.fori_loop(0, L, acc_body, (z, z, z, z),
                                           unroll=4)
            feat_v[g, pl.ds(0, VL)] = a0
            feat_v[g, pl.ds(VL, VL)] = a1
            feat_v[g, pl.ds(2 * VL, VL)] = c0
            feat_v[g, pl.ds(3 * VL, VL)] = c1

        def chunk_body(ch, _):
            rbase = pl.multiple_of(wbase + ch * G, 8)
            pltpu.sync_copy(shp_hbm.at[pl.ds(rbase, G)], idx_s)
            pltpu.sync_copy(col_hbm.at[pl.ds(rbase, G)], idx_c)

            offs = tuple(range(0, L - VL, VL)) + (L - VL,)

            def mask_body(g, _):
                for o in offs:
                    sl = pl.ds(pl.multiple_of(o, 8), VL)
                    s = idx_s[g, sl]
                    c = idx_c[g, sl]
                    idx_c[g, sl] = jnp.where(s == 0, 0, c)
                return 0

            lax.fori_loop(0, G, mask_body, 0)

            issue(0, 0)
            issue(1, 1)

            def pair_body(pp, _):
                for b in range(2):
                    g = pp * 2 + b
                    wait(g, b)
                    accumulate(g, b)
                    issue(g + 2, b)
                return 0

            lax.fori_loop(0, G // 2 - 1, pair_body, 0)
            for b in range(2):
                g = G - 2 + b
                wait(g, b)
                accumulate(g, b)

            pltpu.sync_copy(feat_v, out_hbm.at[pl.ds(wbase + ch * G, G)])
            return 0

        lax.fori_loop(0, NCH, chunk_body, 0)

    return pool(shp_ids, col_ids, shape_table, color_table)


def _mlp(feat_raw, lens_col, W1, b1, W2, b2):
    Bb = 2048

    def body(feat_ref, lens_ref, w1_ref, b1_ref, w2_ref, b2_ref, out_ref):
        f = feat_ref[...]
        inv = 1.0 / lens_ref[...]
        h = jnp.dot(f, w1_ref[...], preferred_element_type=jnp.float32)
        h = jnp.maximum(h * inv + b1_ref[...], 0.0)
        out_ref[...] = (
            jnp.dot(h, w2_ref[...], preferred_element_type=jnp.float32)
            + b2_ref[...]
        )

    return pl.pallas_call(
        body,
        grid=(B // Bb,),
        in_specs=[
            pl.BlockSpec((Bb, 2 * D), lambda i: (i, 0)),
            pl.BlockSpec((Bb, 1), lambda i: (i, 0)),
            pl.BlockSpec((2 * D, H), lambda i: (0, 0)),
            pl.BlockSpec((1, H), lambda i: (0, 0)),
            pl.BlockSpec((H, NLAB), lambda i: (0, 0)),
            pl.BlockSpec((1, NLAB), lambda i: (0, 0)),
        ],
        out_specs=pl.BlockSpec((Bb, NLAB), lambda i: (i, 0)),
        out_shape=jax.ShapeDtypeStruct((B, NLAB), jnp.float32),
    )(feat_raw, lens_col, W1, b1.reshape(1, H), W2, b2.reshape(1, NLAB))


def kernel(shp_ids, col_ids, lens, shape_table, color_table, W1, b1, W2, b2):
    feat = _sc_pool(shp_ids, col_ids, shape_table, color_table)
    lens_col = lens.astype(jnp.float32).reshape(B, 1)
    return _mlp(feat, lens_col, W1, b1, W2, b2)
